# baseline (speedup 1.0000x reference)
import functools

import jax
import jax.numpy as jnp
import numpy as np
from jax import lax
from jax.experimental import pallas as pl
from jax.experimental.pallas import tpu as pltpu
from jax.experimental.pallas import tpu_sc as plsc

N = 10000
E = 320000
F_IN = 128
H = 64
EPS = 1e-5

NC, NS = 2, 16
NW = NC * NS
C = 128
NCHUNK = 80
EPW = C * NCHUNK
EPAD = NW * EPW
NP = 10112
RPS = NP // NS
RT = 2528
GRID = NP // RT


def _sc_mesh():
  return plsc.VectorSubcoreMesh(core_axis_name="c", subcore_axis_name="s")


def _make_prop(F, tc_tiling, G):
  NG = NCHUNK // G
  scratch = [
      pltpu.VMEM((NCHUNK, C), jnp.int32),
      pltpu.VMEM((NCHUNK, C), jnp.int32),
      pltpu.VMEM((2, G, C, F), jnp.float32),
      pltpu.VMEM_SHARED((NP, F), jnp.float32),
      pltpu.SemaphoreType.DMA,
      pltpu.SemaphoreType.DMA,
      pltpu.SemaphoreType.DMA,
      pltpu.SemaphoreType.DMA,
  ]

  @functools.partial(
      pl.kernel,
      out_type=jax.ShapeDtypeStruct((NC, NP, F), jnp.float32),
      mesh=_sc_mesh(),
      scratch_types=scratch,
      compiler_params=None if tc_tiling else pltpu.CompilerParams(
          use_tc_tiling_on_sc=False),
  )
  def k(g_hbm, srcp_hbm, dstp_hbm, zeros_hbm, out_hbm,
        src_v, dst_v, rows_v, acc_sh, sg0, sg1, ss0, ss1):
    sg = (sg0, sg1)
    ss = (ss0, ss1)
    c = lax.axis_index("c")
    s = lax.axis_index("s")
    wid = s * NC + c
    sl = pl.ds(s * RPS, RPS)
    pltpu.sync_copy(zeros_hbm, acc_sh.at[sl])
    pltpu.sync_copy(srcp_hbm.at[wid], src_v)
    pltpu.sync_copy(dstp_hbm.at[wid], dst_v)
    plsc.subcore_barrier()

    def fire_g(i, b):
      for t in range(G):
        pltpu.async_copy(g_hbm.at[src_v.at[i * G + t]], rows_v.at[b, t],
                         sg[b])

    def drain_g(i, b):
      for t in range(G):
        pltpu.make_async_copy(g_hbm.at[src_v.at[i * G + t]], rows_v.at[b, t],
                              sg[b]).wait()

    def fire_s(i, b):
      for t in range(G):
        pltpu.async_copy(rows_v.at[b, t], acc_sh.at[dst_v.at[i * G + t]],
                         ss[b], add=True)

    def drain_s(i, b):
      for t in range(G):
        pltpu.make_async_copy(rows_v.at[b, t], acc_sh.at[dst_v.at[i * G + t]],
                              ss[b]).wait()

    fire_g(0, 0)

    def outer(k2, carry):
      for b in (0, 1):
        i = k2 * 2 + b

        @pl.when(i > 0)
        def _():
          drain_s(i - 1, 1 - b)

        @pl.when(i + 1 < NG)
        def _():
          fire_g(i + 1, 1 - b)

        drain_g(i, b)
        fire_s(i, b)
      return carry

    lax.fori_loop(0, NG // 2, outer, 0)
    drain_s(NG - 1, 1)
    plsc.subcore_barrier()
    pltpu.sync_copy(acc_sh.at[sl], out_hbm.at[c, sl])

  return k


_prop64 = _make_prop(H, tc_tiling=False, G=4)
_prop16 = _make_prop(16, tc_tiling=False, G=4)


def _tc_specs(shapes):
  specs = []
  for sh in shapes:
    if sh[0] == NP:
      blk = (RT,) + sh[1:]
      specs.append(
          pl.BlockSpec(blk, lambda i, _n=len(sh): (i,) + (0,) * (_n - 1)))
    elif sh[0] == NC and len(sh) == 3:
      specs.append(pl.BlockSpec((NC, RT, sh[2]), lambda i: (0, i, 0)))
    else:
      specs.append(pl.BlockSpec(sh, lambda i, _n=len(sh): (0,) * _n))
  return specs


def _tc_call(body, in_arrays, out_shapes):
  in_specs = _tc_specs([a.shape for a in in_arrays])
  out_specs = _tc_specs([s.shape for s in out_shapes])
  return pl.pallas_call(
      body,
      grid=(GRID,),
      in_specs=in_specs,
      out_specs=out_specs if len(out_specs) > 1 else out_specs[0],
      out_shape=out_shapes if len(out_shapes) > 1 else out_shapes[0],
  )(*in_arrays)


def _k_pre(dp_ref, x_ref, w_ref, b_ref, dinv_ref, qa_ref, qb_ref, a0_ref):
  deg = dp_ref[0] + dp_ref[1]
  dinv = jnp.where(deg > 0.0, lax.rsqrt(jnp.maximum(deg, 1e-30)), 0.0)
  dinv_ref[...] = dinv
  x = x_ref[...]
  qa_ref[...] = dinv * x[:, :H]
  qb_ref[...] = dinv * x[:, H:]
  a0_ref[...] = (
      jnp.dot(x, w_ref[...], preferred_element_type=jnp.float32)
      + b_ref[...])


def _k_mid128(sa_ref, sb_ref, dinv_ref, a_ref, wa_ref, wb_ref,
              qa_ref, qb_ref, a1_ref):
  dinv = dinv_ref[...]
  ta = -dinv * (sa_ref[0] + sa_ref[1])
  tb = -dinv * (sb_ref[0] + sb_ref[1])
  qa_ref[...] = dinv * ta
  qb_ref[...] = dinv * tb
  a1_ref[...] = (
      a_ref[...]
      + jnp.dot(ta, wa_ref[...], preferred_element_type=jnp.float32)
      + jnp.dot(tb, wb_ref[...], preferred_element_type=jnp.float32))


def _k_post128(sa_ref, sb_ref, dinv_ref, a_ref, hin_ref, wa_ref, wb_ref,
               sc_ref, be_ref, wn_ref, bn_ref, h_ref, qn_ref, an_ref):
  dinv = dinv_ref[...]
  hx = hin_ref[...]
  ta = -2.0 * dinv * (sa_ref[0] + sa_ref[1]) - hx[:, :H]
  tb = -2.0 * dinv * (sb_ref[0] + sb_ref[1]) - hx[:, H:]
  out = (a_ref[...]
         + jnp.dot(ta, wa_ref[...], preferred_element_type=jnp.float32)
         + jnp.dot(tb, wb_ref[...], preferred_element_type=jnp.float32))
  out = out * sc_ref[...] + be_ref[...]
  h = jnp.maximum(out, 0.0)
  h_ref[...] = h
  qn_ref[...] = dinv * h
  an_ref[...] = (
      jnp.dot(h, wn_ref[...], preferred_element_type=jnp.float32)
      + bn_ref[...])


def _k_mid(sp_ref, dinv_ref, a_ref, w_ref, q1_ref, a1_ref):
  dinv = dinv_ref[...]
  tx1 = -dinv * (sp_ref[0] + sp_ref[1])
  q1_ref[...] = dinv * tx1
  a1_ref[...] = a_ref[...] + jnp.dot(
      tx1, w_ref[...], preferred_element_type=jnp.float32)


def _k_post(has_res):
  def body(sp_ref, dinv_ref, a_ref, hin_ref, w2_ref, sc_ref, be_ref,
           wn_ref, bn_ref, h_ref, qn_ref, an_ref):
    dinv = dinv_ref[...]
    tx2 = -2.0 * dinv * (sp_ref[0] + sp_ref[1]) - hin_ref[...]
    out = a_ref[...] + jnp.dot(
        tx2, w2_ref[...], preferred_element_type=jnp.float32)
    out = out * sc_ref[...] + be_ref[...]
    h = jnp.maximum(out, 0.0)
    if has_res:
      h = h + hin_ref[...]
    h_ref[...] = h
    qn_ref[...] = dinv * h
    an_ref[...] = (
        jnp.dot(h, wn_ref[...], preferred_element_type=jnp.float32)
        + bn_ref[...])
  return body


def _k_fin(sp_ref, dinv_ref, a_ref, hin_ref, w2_ref, sc_ref, be_ref,
           wo_ref, bo_ref, out_ref):
  dinv = dinv_ref[...]
  tx2 = -2.0 * dinv * (sp_ref[0] + sp_ref[1]) - hin_ref[...]
  out = a_ref[...] + jnp.dot(
      tx2, w2_ref[...], preferred_element_type=jnp.float32)
  out = out * sc_ref[...] + be_ref[...]
  h = jnp.maximum(out, 0.0) + hin_ref[...]
  out_ref[...] = (
      jnp.dot(h, wo_ref[...], preferred_element_type=jnp.float32)
      + bo_ref[...])


def _sds(*shape):
  return jax.ShapeDtypeStruct(shape, jnp.float32)


@jax.jit
def kernel(x, edge_index, W0, b0, g0, be0, W1, b1, g1, be1, W2, b2, g2, be2,
           Wout, bout):
  f32 = jnp.float32
  src = edge_index[0]
  dst = edge_index[1]
  pad = jnp.full((EPAD - E,), N, dtype=jnp.int32)
  srcp = jnp.concatenate([src, pad]).reshape(NW, NCHUNK, C)
  dstp = jnp.concatenate([dst, pad]).reshape(NW, NCHUNK, C)
  x_p = jnp.concatenate([x, jnp.zeros((NP - N, F_IN), f32)], axis=0)

  z64 = jnp.zeros((RPS, H), f32)
  ones_t = jnp.ones((NP, 16), f32)
  z16 = jnp.zeros((RPS, 16), f32)

  inv_bn = np.float32(1.0 / np.sqrt(1.0 + EPS))
  s0 = (g0 * inv_bn).reshape(1, H)
  s1 = (g1 * inv_bn).reshape(1, H)
  s2 = (g2 * inv_bn).reshape(1, H)
  be0r, be1r, be2r = be0.reshape(1, H), be1.reshape(1, H), be2.reshape(1, H)
  b0r, b1r, b2r = b0.reshape(1, H), b1.reshape(1, H), b2.reshape(1, H)
  boutr = bout.reshape(1, 2)

  dp = _prop16(ones_t, srcp, srcp, z16)[:, :, :1]
  dinv, qa, qb, a0 = _tc_call(
      _k_pre, [dp, x_p, W0[0], b0r],
      [_sds(NP, 1), _sds(NP, H), _sds(NP, H), _sds(NP, H)])

  sa = _prop64(qa, srcp, dstp, z64)
  sb = _prop64(qb, srcp, dstp, z64)
  q1a, q1b, a1 = _tc_call(
      _k_mid128, [sa, sb, dinv, a0, W0[1][:H], W0[1][H:]],
      [_sds(NP, H), _sds(NP, H), _sds(NP, H)])
  s2a = _prop64(q1a, srcp, dstp, z64)
  s2b = _prop64(q1b, srcp, dstp, z64)
  h1, qn, an = _tc_call(
      _k_post128, [s2a, s2b, dinv, a1, x_p, W0[2][:H], W0[2][H:],
                   s0, be0r, W1[0], b1r],
      [_sds(NP, H), _sds(NP, H), _sds(NP, H)])

  sC = _prop64(qn, srcp, dstp, z64)
  q1b, a1b = _tc_call(_k_mid, [sC, dinv, an, W1[1]],
                      [_sds(NP, H), _sds(NP, H)])
  sD = _prop64(q1b, srcp, dstp, z64)
  h2, qc, ac = _tc_call(
      _k_post(True), [sD, dinv, a1b, h1, W1[2], s1, be1r, W2[0], b2r],
      [_sds(NP, H), _sds(NP, H), _sds(NP, H)])

  sE = _prop64(qc, srcp, dstp, z64)
  q1c, a1c = _tc_call(_k_mid, [sE, dinv, ac, W2[1]],
                      [_sds(NP, H), _sds(NP, H)])
  sF = _prop64(q1c, srcp, dstp, z64)
  coords_p = _tc_call(
      _k_fin, [sF, dinv, a1c, h2, W2[2], s2, be2r, Wout, boutr],
      [_sds(NP, 2)])
  return coords_p[:N]


# speedup vs baseline: 3.1085x; 3.1085x over previous
import functools

import jax
import jax.numpy as jnp
import numpy as np
from jax import lax
from jax.experimental import pallas as pl
from jax.experimental.pallas import tpu as pltpu
from jax.experimental.pallas import tpu_sc as plsc

N = 10000
E = 320000
F_IN = 128
H = 64
EPS = 1e-5

NC, NS = 2, 16
NW = NC * NS
C = 128
NCHUNK = 80
EPW = C * NCHUNK
EPAD = NW * EPW
NP = 10112
RPS = NP // NS
RT = 2528
GRID = NP // RT


def _sc_mesh():
  return plsc.VectorSubcoreMesh(core_axis_name="c", subcore_axis_name="s")


def _make_prop(F, tc_tiling, G):
  NG = NCHUNK // G
  scratch = [
      pltpu.VMEM((NCHUNK, C), jnp.int32),
      pltpu.VMEM((NCHUNK, C), jnp.int32),
      pltpu.VMEM((2, G, C, F), jnp.float32),
      pltpu.VMEM_SHARED((NP, F), jnp.float32),
      pltpu.SemaphoreType.DMA,
      pltpu.SemaphoreType.DMA,
      pltpu.SemaphoreType.DMA,
      pltpu.SemaphoreType.DMA,
  ]

  @functools.partial(
      pl.kernel,
      out_type=jax.ShapeDtypeStruct((NC, NP, F), jnp.float32),
      mesh=_sc_mesh(),
      scratch_types=scratch,
      compiler_params=None if tc_tiling else pltpu.CompilerParams(
          use_tc_tiling_on_sc=False),
  )
  def k(g_hbm, srcp_hbm, dstp_hbm, zeros_hbm, out_hbm,
        src_v, dst_v, rows_v, acc_sh, sg0, sg1, ss0, ss1):
    sg = (sg0, sg1)
    ss = (ss0, ss1)
    c = lax.axis_index("c")
    s = lax.axis_index("s")
    wid = s * NC + c
    sl = pl.ds(s * RPS, RPS)
    pltpu.sync_copy(zeros_hbm, acc_sh.at[sl])
    pltpu.sync_copy(srcp_hbm.at[wid], src_v)
    pltpu.sync_copy(dstp_hbm.at[wid], dst_v)
    plsc.subcore_barrier()

    def fire_g(i, b):
      for t in range(G):
        pltpu.async_copy(g_hbm.at[src_v.at[i * G + t]], rows_v.at[b, t],
                         sg[b])

    def drain_g(i, b):
      for t in range(G):
        pltpu.make_async_copy(g_hbm.at[src_v.at[i * G + t]], rows_v.at[b, t],
                              sg[b]).wait()

    def fire_s(i, b):
      for t in range(G):
        pltpu.async_copy(rows_v.at[b, t], acc_sh.at[dst_v.at[i * G + t]],
                         ss[b], add=True)

    def drain_s(i, b):
      for t in range(G):
        pltpu.make_async_copy(rows_v.at[b, t], acc_sh.at[dst_v.at[i * G + t]],
                              ss[b]).wait()

    fire_g(0, 0)

    def outer(k2, carry):
      for b in (0, 1):
        i = k2 * 2 + b

        @pl.when(i > 0)
        def _():
          drain_s(i - 1, 1 - b)

        @pl.when(i + 1 < NG)
        def _():
          fire_g(i + 1, 1 - b)

        drain_g(i, b)
        fire_s(i, b)
      return carry

    lax.fori_loop(0, NG // 2, outer, 0)
    drain_s(NG - 1, 1)
    plsc.subcore_barrier()
    pltpu.sync_copy(acc_sh.at[sl], out_hbm.at[c, sl])

  return k


_prop64 = _make_prop(H, tc_tiling=False, G=4)
_prop16 = _make_prop(16, tc_tiling=False, G=4)


def _tc_specs(shapes):
  specs = []
  for sh in shapes:
    if sh[0] == NP:
      blk = (RT,) + sh[1:]
      specs.append(
          pl.BlockSpec(blk, lambda i, _n=len(sh): (i,) + (0,) * (_n - 1)))
    elif sh[0] == NC and len(sh) == 3:
      specs.append(pl.BlockSpec((NC, RT, sh[2]), lambda i: (0, i, 0)))
    else:
      specs.append(pl.BlockSpec(sh, lambda i, _n=len(sh): (0,) * _n))
  return specs


def _tc_call(body, in_arrays, out_shapes):
  in_specs = _tc_specs([a.shape for a in in_arrays])
  out_specs = _tc_specs([s.shape for s in out_shapes])
  return pl.pallas_call(
      body,
      grid=(GRID,),
      in_specs=in_specs,
      out_specs=out_specs if len(out_specs) > 1 else out_specs[0],
      out_shape=out_shapes if len(out_shapes) > 1 else out_shapes[0],
  )(*in_arrays)


def _k_pre(dp_ref, x_ref, w_ref, b_ref, dinv_ref, qa_ref, qb_ref, a0_ref):
  deg = dp_ref[0] + dp_ref[1]
  dinv = jnp.where(deg > 0.0, lax.rsqrt(jnp.maximum(deg, 1e-30)), 0.0)
  dinv_ref[...] = dinv
  x = x_ref[...]
  qa_ref[...] = dinv * x[:, :H]
  qb_ref[...] = dinv * x[:, H:]
  a0_ref[...] = (
      jnp.dot(x, w_ref[...], preferred_element_type=jnp.float32)
      + b_ref[...])


def _k_mid128(sa_ref, sb_ref, dinv_ref, a_ref, wa_ref, wb_ref,
              qa_ref, qb_ref, a1_ref):
  dinv = dinv_ref[...]
  ta = -dinv * (sa_ref[0] + sa_ref[1])
  tb = -dinv * (sb_ref[0] + sb_ref[1])
  qa_ref[...] = dinv * ta
  qb_ref[...] = dinv * tb
  a1_ref[...] = (
      a_ref[...]
      + jnp.dot(ta, wa_ref[...], preferred_element_type=jnp.float32)
      + jnp.dot(tb, wb_ref[...], preferred_element_type=jnp.float32))


def _k_post128(sa_ref, sb_ref, dinv_ref, a_ref, hin_ref, wa_ref, wb_ref,
               sc_ref, be_ref, wn_ref, bn_ref, h_ref, qn_ref, an_ref):
  dinv = dinv_ref[...]
  hx = hin_ref[...]
  ta = -2.0 * dinv * (sa_ref[0] + sa_ref[1]) - hx[:, :H]
  tb = -2.0 * dinv * (sb_ref[0] + sb_ref[1]) - hx[:, H:]
  out = (a_ref[...]
         + jnp.dot(ta, wa_ref[...], preferred_element_type=jnp.float32)
         + jnp.dot(tb, wb_ref[...], preferred_element_type=jnp.float32))
  out = out * sc_ref[...] + be_ref[...]
  h = jnp.maximum(out, 0.0)
  h_ref[...] = h
  qn_ref[...] = dinv * h
  an_ref[...] = (
      jnp.dot(h, wn_ref[...], preferred_element_type=jnp.float32)
      + bn_ref[...])


def _k_mid(sp_ref, dinv_ref, a_ref, w_ref, q1_ref, a1_ref):
  dinv = dinv_ref[...]
  tx1 = -dinv * (sp_ref[0] + sp_ref[1])
  q1_ref[...] = dinv * tx1
  a1_ref[...] = a_ref[...] + jnp.dot(
      tx1, w_ref[...], preferred_element_type=jnp.float32)


def _k_post(has_res):
  def body(sp_ref, dinv_ref, a_ref, hin_ref, w2_ref, sc_ref, be_ref,
           wn_ref, bn_ref, h_ref, qn_ref, an_ref):
    dinv = dinv_ref[...]
    tx2 = -2.0 * dinv * (sp_ref[0] + sp_ref[1]) - hin_ref[...]
    out = a_ref[...] + jnp.dot(
        tx2, w2_ref[...], preferred_element_type=jnp.float32)
    out = out * sc_ref[...] + be_ref[...]
    h = jnp.maximum(out, 0.0)
    if has_res:
      h = h + hin_ref[...]
    h_ref[...] = h
    qn_ref[...] = dinv * h
    an_ref[...] = (
        jnp.dot(h, wn_ref[...], preferred_element_type=jnp.float32)
        + bn_ref[...])
  return body


def _k_fin(sp_ref, dinv_ref, a_ref, hin_ref, w2_ref, sc_ref, be_ref,
           wo_ref, bo_ref, out_ref):
  dinv = dinv_ref[...]
  tx2 = -2.0 * dinv * (sp_ref[0] + sp_ref[1]) - hin_ref[...]
  out = a_ref[...] + jnp.dot(
      tx2, w2_ref[...], preferred_element_type=jnp.float32)
  out = out * sc_ref[...] + be_ref[...]
  h = jnp.maximum(out, 0.0) + hin_ref[...]
  out_ref[...] = (
      jnp.dot(h, wo_ref[...], preferred_element_type=jnp.float32)
      + bo_ref[...])


def _sds(*shape):
  return jax.ShapeDtypeStruct(shape, jnp.float32)


@jax.jit
def kernel(x, edge_index, W0, b0, g0, be0, W1, b1, g1, be1, W2, b2, g2, be2,
           Wout, bout):
  f32 = jnp.float32
  src = edge_index[0]
  dst = edge_index[1]
  ppw = EPW - E // NW
  pad = N + (np.arange(NW * ppw, dtype=np.int32) % (NP - N)).reshape(NW, ppw)
  pad = jnp.asarray(pad)
  srcp = jnp.concatenate(
      [src.reshape(NW, E // NW), pad], axis=1).reshape(NW, NCHUNK, C)
  dstp = jnp.concatenate(
      [dst.reshape(NW, E // NW), pad], axis=1).reshape(NW, NCHUNK, C)
  x_p = jnp.concatenate([x, jnp.zeros((NP - N, F_IN), f32)], axis=0)

  z64 = jnp.zeros((RPS, H), f32)
  ones_t = jnp.ones((NP, 16), f32)
  z16 = jnp.zeros((RPS, 16), f32)

  inv_bn = np.float32(1.0 / np.sqrt(1.0 + EPS))
  s0 = (g0 * inv_bn).reshape(1, H)
  s1 = (g1 * inv_bn).reshape(1, H)
  s2 = (g2 * inv_bn).reshape(1, H)
  be0r, be1r, be2r = be0.reshape(1, H), be1.reshape(1, H), be2.reshape(1, H)
  b0r, b1r, b2r = b0.reshape(1, H), b1.reshape(1, H), b2.reshape(1, H)
  boutr = bout.reshape(1, 2)

  dp = _prop16(ones_t, srcp, srcp, z16)[:, :, :1]
  dinv, qa, qb, a0 = _tc_call(
      _k_pre, [dp, x_p, W0[0], b0r],
      [_sds(NP, 1), _sds(NP, H), _sds(NP, H), _sds(NP, H)])

  sa = _prop64(qa, srcp, dstp, z64)
  sb = _prop64(qb, srcp, dstp, z64)
  q1a, q1b, a1 = _tc_call(
      _k_mid128, [sa, sb, dinv, a0, W0[1][:H], W0[1][H:]],
      [_sds(NP, H), _sds(NP, H), _sds(NP, H)])
  s2a = _prop64(q1a, srcp, dstp, z64)
  s2b = _prop64(q1b, srcp, dstp, z64)
  h1, qn, an = _tc_call(
      _k_post128, [s2a, s2b, dinv, a1, x_p, W0[2][:H], W0[2][H:],
                   s0, be0r, W1[0], b1r],
      [_sds(NP, H), _sds(NP, H), _sds(NP, H)])

  sC = _prop64(qn, srcp, dstp, z64)
  q1b, a1b = _tc_call(_k_mid, [sC, dinv, an, W1[1]],
                      [_sds(NP, H), _sds(NP, H)])
  sD = _prop64(q1b, srcp, dstp, z64)
  h2, qc, ac = _tc_call(
      _k_post(True), [sD, dinv, a1b, h1, W1[2], s1, be1r, W2[0], b2r],
      [_sds(NP, H), _sds(NP, H), _sds(NP, H)])

  sE = _prop64(qc, srcp, dstp, z64)
  q1c, a1c = _tc_call(_k_mid, [sE, dinv, ac, W2[1]],
                      [_sds(NP, H), _sds(NP, H)])
  sF = _prop64(q1c, srcp, dstp, z64)
  coords_p = _tc_call(
      _k_fin, [sF, dinv, a1c, h2, W2[2], s2, be2r, Wout, boutr],
      [_sds(NP, 2)])
  return coords_p[:N]


# speedup vs baseline: 3.1455x; 1.0119x over previous
import functools

import jax
import jax.numpy as jnp
import numpy as np
from jax import lax
from jax.experimental import pallas as pl
from jax.experimental.pallas import tpu as pltpu
from jax.experimental.pallas import tpu_sc as plsc

N = 10000
E = 320000
F_IN = 128
H = 64
EPS = 1e-5

NC, NS = 2, 16
NW = NC * NS
C = 128
NCHUNK = 80
EPW = C * NCHUNK
EPAD = NW * EPW
NP = 10112
RPS = NP // NS
RT = 2528
GRID = NP // RT


def _sc_mesh():
  return plsc.VectorSubcoreMesh(core_axis_name="c", subcore_axis_name="s")


def _make_prop(F, tc_tiling, G):
  NG = NCHUNK // G
  scratch = [
      pltpu.VMEM((NCHUNK, C), jnp.int32),
      pltpu.VMEM((NCHUNK, C), jnp.int32),
      pltpu.VMEM((2, G, C, F), jnp.float32),
      pltpu.VMEM_SHARED((NP, F), jnp.float32),
      pltpu.SemaphoreType.DMA,
      pltpu.SemaphoreType.DMA,
      pltpu.SemaphoreType.DMA,
      pltpu.SemaphoreType.DMA,
  ]

  @functools.partial(
      pl.kernel,
      out_type=jax.ShapeDtypeStruct((NC, NP, F), jnp.float32),
      mesh=_sc_mesh(),
      scratch_types=scratch,
      compiler_params=None if tc_tiling else pltpu.CompilerParams(
          use_tc_tiling_on_sc=False),
  )
  def k(g_hbm, srcp_hbm, dstp_hbm, zeros_hbm, out_hbm,
        src_v, dst_v, rows_v, acc_sh, sg0, sg1, ss0, ss1):
    sg = (sg0, sg1)
    ss = (ss0, ss1)
    c = lax.axis_index("c")
    s = lax.axis_index("s")
    wid = s * NC + c
    sl = pl.ds(s * RPS, RPS)
    pltpu.sync_copy(zeros_hbm, acc_sh.at[sl])
    pltpu.sync_copy(srcp_hbm.at[wid], src_v)
    pltpu.sync_copy(dstp_hbm.at[wid], dst_v)
    plsc.subcore_barrier()

    def fire_g(i, b):
      for t in range(G):
        pltpu.async_copy(g_hbm.at[src_v.at[i * G + t]], rows_v.at[b, t],
                         sg[b])

    def drain_g(i, b):
      for t in range(G):
        pltpu.make_async_copy(g_hbm.at[src_v.at[i * G + t]], rows_v.at[b, t],
                              sg[b]).wait()

    def fire_s(i, b):
      for t in range(G):
        pltpu.async_copy(rows_v.at[b, t], acc_sh.at[dst_v.at[i * G + t]],
                         ss[b], add=True)

    def drain_s(i, b):
      for t in range(G):
        pltpu.make_async_copy(rows_v.at[b, t], acc_sh.at[dst_v.at[i * G + t]],
                              ss[b]).wait()

    fire_g(0, 0)

    def outer(k2, carry):
      for b in (0, 1):
        i = k2 * 2 + b

        @pl.when(i > 0)
        def _():
          drain_s(i - 1, 1 - b)

        @pl.when(i + 1 < NG)
        def _():
          fire_g(i + 1, 1 - b)

        drain_g(i, b)
        fire_s(i, b)
      return carry

    lax.fori_loop(0, NG // 2, outer, 0)
    drain_s(NG - 1, 1)
    plsc.subcore_barrier()
    pltpu.sync_copy(acc_sh.at[sl], out_hbm.at[c, sl])

  return k


def _make_prop2(G):
  NG = NCHUNK // G
  scratch = [
      pltpu.VMEM((NCHUNK, C), jnp.int32),
      pltpu.VMEM((NCHUNK, C), jnp.int32),
      pltpu.VMEM((2, G, C, H), jnp.float32),
      pltpu.VMEM_SHARED((NP, H), jnp.float32),
      pltpu.SemaphoreType.DMA,
      pltpu.SemaphoreType.DMA,
      pltpu.SemaphoreType.DMA,
      pltpu.SemaphoreType.DMA,
  ]

  @functools.partial(
      pl.kernel,
      out_type=jax.ShapeDtypeStruct((NC, 2, NP, H), jnp.float32),
      mesh=_sc_mesh(),
      scratch_types=scratch,
      compiler_params=pltpu.CompilerParams(use_tc_tiling_on_sc=False),
  )
  def k(g_hbm, srcp_hbm, dstp_hbm, zeros_hbm, out_hbm,
        src_v, dst_v, rows_v, acc_sh, sg0, sg1, ss0, ss1):
    sg = (sg0, sg1)
    ss = (ss0, ss1)
    c = lax.axis_index("c")
    s = lax.axis_index("s")
    wid = s * NC + c
    sl = pl.ds(s * RPS, RPS)
    pltpu.sync_copy(zeros_hbm, acc_sh.at[sl])
    pltpu.sync_copy(srcp_hbm.at[wid], src_v)
    pltpu.sync_copy(dstp_hbm.at[wid], dst_v)

    for tbl in (0, 1):
      g_t = g_hbm.at[tbl]

      def fire_g(i, b):
        for t in range(G):
          pltpu.async_copy(g_t.at[src_v.at[i * G + t]], rows_v.at[b, t],
                           sg[b])

      def drain_g(i, b):
        for t in range(G):
          pltpu.make_async_copy(g_t.at[src_v.at[i * G + t]], rows_v.at[b, t],
                                sg[b]).wait()

      def fire_s(i, b):
        for t in range(G):
          pltpu.async_copy(rows_v.at[b, t], acc_sh.at[dst_v.at[i * G + t]],
                           ss[b], add=True)

      def drain_s(i, b):
        for t in range(G):
          pltpu.make_async_copy(rows_v.at[b, t],
                                acc_sh.at[dst_v.at[i * G + t]], ss[b]).wait()

      plsc.subcore_barrier()
      fire_g(0, 0)

      def outer(k2, carry):
        for b in (0, 1):
          i = k2 * 2 + b

          @pl.when(i > 0)
          def _():
            drain_s(i - 1, 1 - b)

          @pl.when(i + 1 < NG)
          def _():
            fire_g(i + 1, 1 - b)

          drain_g(i, b)
          fire_s(i, b)
        return carry

      lax.fori_loop(0, NG // 2, outer, 0)
      drain_s(NG - 1, 1)
      plsc.subcore_barrier()
      pltpu.sync_copy(acc_sh.at[sl], out_hbm.at[c, tbl, sl])
      if tbl == 0:
        pltpu.sync_copy(zeros_hbm, acc_sh.at[sl])

  return k

_prop64 = _make_prop(H, tc_tiling=False, G=4)
_prop64x2 = _make_prop2(G=4)
_prop16 = _make_prop(16, tc_tiling=False, G=4)


def _tc_specs(shapes):
  specs = []
  for sh in shapes:
    if sh[0] == NP:
      blk = (RT,) + sh[1:]
      specs.append(
          pl.BlockSpec(blk, lambda i, _n=len(sh): (i,) + (0,) * (_n - 1)))
    elif sh[0] == NC and len(sh) == 3:
      specs.append(pl.BlockSpec((NC, RT, sh[2]), lambda i: (0, i, 0)))
    elif sh[0] == NC and len(sh) == 4:
      specs.append(
          pl.BlockSpec((NC, 2, RT, sh[3]), lambda i: (0, 0, i, 0)))
    else:
      specs.append(pl.BlockSpec(sh, lambda i, _n=len(sh): (0,) * _n))
  return specs


def _tc_call(body, in_arrays, out_shapes):
  in_specs = _tc_specs([a.shape for a in in_arrays])
  out_specs = _tc_specs([s.shape for s in out_shapes])
  return pl.pallas_call(
      body,
      grid=(GRID,),
      in_specs=in_specs,
      out_specs=out_specs if len(out_specs) > 1 else out_specs[0],
      out_shape=out_shapes if len(out_shapes) > 1 else out_shapes[0],
  )(*in_arrays)


def _k_pre(dp_ref, x_ref, w_ref, b_ref, dinv_ref, qab_ref, a0_ref):
  deg = dp_ref[0] + dp_ref[1]
  dinv = jnp.where(deg > 0.0, lax.rsqrt(jnp.maximum(deg, 1e-30)), 0.0)
  dinv_ref[...] = dinv
  x = x_ref[...]
  qab_ref[0] = dinv * x[:, :H]
  qab_ref[1] = dinv * x[:, H:]
  a0_ref[...] = (
      jnp.dot(x, w_ref[...], preferred_element_type=jnp.float32)
      + b_ref[...])


def _k_mid128(sab_ref, dinv_ref, a_ref, wa_ref, wb_ref,
              q1ab_ref, a1_ref):
  dinv = dinv_ref[...]
  ta = -dinv * (sab_ref[0, 0] + sab_ref[1, 0])
  tb = -dinv * (sab_ref[0, 1] + sab_ref[1, 1])
  q1ab_ref[0] = dinv * ta
  q1ab_ref[1] = dinv * tb
  a1_ref[...] = (
      a_ref[...]
      + jnp.dot(ta, wa_ref[...], preferred_element_type=jnp.float32)
      + jnp.dot(tb, wb_ref[...], preferred_element_type=jnp.float32))


def _k_post128(sab_ref, dinv_ref, a_ref, hin_ref, wa_ref, wb_ref,
               sc_ref, be_ref, wn_ref, bn_ref, h_ref, qn_ref, an_ref):
  dinv = dinv_ref[...]
  hx = hin_ref[...]
  ta = -2.0 * dinv * (sab_ref[0, 0] + sab_ref[1, 0]) - hx[:, :H]
  tb = -2.0 * dinv * (sab_ref[0, 1] + sab_ref[1, 1]) - hx[:, H:]
  out = (a_ref[...]
         + jnp.dot(ta, wa_ref[...], preferred_element_type=jnp.float32)
         + jnp.dot(tb, wb_ref[...], preferred_element_type=jnp.float32))
  out = out * sc_ref[...] + be_ref[...]
  h = jnp.maximum(out, 0.0)
  h_ref[...] = h
  qn_ref[...] = dinv * h
  an_ref[...] = (
      jnp.dot(h, wn_ref[...], preferred_element_type=jnp.float32)
      + bn_ref[...])


def _k_mid(sp_ref, dinv_ref, a_ref, w_ref, q1_ref, a1_ref):
  dinv = dinv_ref[...]
  tx1 = -dinv * (sp_ref[0] + sp_ref[1])
  q1_ref[...] = dinv * tx1
  a1_ref[...] = a_ref[...] + jnp.dot(
      tx1, w_ref[...], preferred_element_type=jnp.float32)


def _k_post(has_res):
  def body(sp_ref, dinv_ref, a_ref, hin_ref, w2_ref, sc_ref, be_ref,
           wn_ref, bn_ref, h_ref, qn_ref, an_ref):
    dinv = dinv_ref[...]
    tx2 = -2.0 * dinv * (sp_ref[0] + sp_ref[1]) - hin_ref[...]
    out = a_ref[...] + jnp.dot(
        tx2, w2_ref[...], preferred_element_type=jnp.float32)
    out = out * sc_ref[...] + be_ref[...]
    h = jnp.maximum(out, 0.0)
    if has_res:
      h = h + hin_ref[...]
    h_ref[...] = h
    qn_ref[...] = dinv * h
    an_ref[...] = (
        jnp.dot(h, wn_ref[...], preferred_element_type=jnp.float32)
        + bn_ref[...])
  return body


def _k_fin(sp_ref, dinv_ref, a_ref, hin_ref, w2_ref, sc_ref, be_ref,
           wo_ref, bo_ref, out_ref):
  dinv = dinv_ref[...]
  tx2 = -2.0 * dinv * (sp_ref[0] + sp_ref[1]) - hin_ref[...]
  out = a_ref[...] + jnp.dot(
      tx2, w2_ref[...], preferred_element_type=jnp.float32)
  out = out * sc_ref[...] + be_ref[...]
  h = jnp.maximum(out, 0.0) + hin_ref[...]
  out_ref[...] = (
      jnp.dot(h, wo_ref[...], preferred_element_type=jnp.float32)
      + bo_ref[...])


def _sds(*shape):
  return jax.ShapeDtypeStruct(shape, jnp.float32)


@jax.jit
def kernel(x, edge_index, W0, b0, g0, be0, W1, b1, g1, be1, W2, b2, g2, be2,
           Wout, bout):
  f32 = jnp.float32
  src = edge_index[0]
  dst = edge_index[1]
  ppw = EPW - E // NW
  pad = N + (np.arange(NW * ppw, dtype=np.int32) % (NP - N)).reshape(NW, ppw)
  pad = jnp.asarray(pad)
  srcp = jnp.concatenate(
      [src.reshape(NW, E // NW), pad], axis=1).reshape(NW, NCHUNK, C)
  dstp = jnp.concatenate(
      [dst.reshape(NW, E // NW), pad], axis=1).reshape(NW, NCHUNK, C)
  x_p = jnp.concatenate([x, jnp.zeros((NP - N, F_IN), f32)], axis=0)

  z64 = jnp.zeros((RPS, H), f32)
  ones_t = jnp.ones((NP, 16), f32)
  z16 = jnp.zeros((RPS, 16), f32)

  inv_bn = np.float32(1.0 / np.sqrt(1.0 + EPS))
  s0 = (g0 * inv_bn).reshape(1, H)
  s1 = (g1 * inv_bn).reshape(1, H)
  s2 = (g2 * inv_bn).reshape(1, H)
  be0r, be1r, be2r = be0.reshape(1, H), be1.reshape(1, H), be2.reshape(1, H)
  b0r, b1r, b2r = b0.reshape(1, H), b1.reshape(1, H), b2.reshape(1, H)
  boutr = bout.reshape(1, 2)

  dp = _prop16(ones_t, srcp, srcp, z16)[:, :, :1]
  dinv, qab, a0 = _tc_call(
      _k_pre, [dp, x_p, W0[0], b0r],
      [_sds(NP, 1), _sds(NC, NP, H), _sds(NP, H)])

  sab = _prop64x2(qab, srcp, dstp, z64)
  q1ab, a1 = _tc_call(
      _k_mid128, [sab, dinv, a0, W0[1][:H], W0[1][H:]],
      [_sds(NC, NP, H), _sds(NP, H)])
  s2ab = _prop64x2(q1ab, srcp, dstp, z64)
  h1, qn, an = _tc_call(
      _k_post128, [s2ab, dinv, a1, x_p, W0[2][:H], W0[2][H:],
                   s0, be0r, W1[0], b1r],
      [_sds(NP, H), _sds(NP, H), _sds(NP, H)])

  sC = _prop64(qn, srcp, dstp, z64)
  q1b, a1b = _tc_call(_k_mid, [sC, dinv, an, W1[1]],
                      [_sds(NP, H), _sds(NP, H)])
  sD = _prop64(q1b, srcp, dstp, z64)
  h2, qc, ac = _tc_call(
      _k_post(True), [sD, dinv, a1b, h1, W1[2], s1, be1r, W2[0], b2r],
      [_sds(NP, H), _sds(NP, H), _sds(NP, H)])

  sE = _prop64(qc, srcp, dstp, z64)
  q1c, a1c = _tc_call(_k_mid, [sE, dinv, ac, W2[1]],
                      [_sds(NP, H), _sds(NP, H)])
  sF = _prop64(q1c, srcp, dstp, z64)
  coords_p = _tc_call(
      _k_fin, [sF, dinv, a1c, h2, W2[2], s2, be2r, Wout, boutr],
      [_sds(NP, 2)])
  return coords_p[:N]
